# baseline (device time: 131397 ns/iter reference)
import jax
import jax.numpy as jnp
from jax import lax
from jax.experimental import pallas as pl
from jax.experimental.pallas import tpu as pltpu

NBITS = 13
SUB = 256
SUB_LOG2 = 8


def kernel(x, dest):
    T, D = x.shape
    R = D // 128
    my_y_out = lax.axis_index("y")

    send_mask = (dest != my_y_out).astype(jnp.int32)
    perm = jnp.argsort(send_mask, stable=True).astype(jnp.int32)
    n_keep_arr = jnp.sum(1 - send_mask).astype(jnp.int32).reshape(1)

    x3 = x.reshape(T, R, 128)

    def body(n_keep_ref, perm_ref, x_ref, out_ref, send_buf, send_sems, recv_sems):
        my_x = lax.axis_index("x")
        my_y = lax.axis_index("y")
        my_z = lax.axis_index("z")
        peer = (my_x, 1 - my_y, my_z)

        n_keep = n_keep_ref[0]
        n_move = T - n_keep

        keep_base = jnp.where(my_y == 0, 0, n_move)
        dst_base = jnp.where(my_y == 0, 0, n_keep)
        recv_base = jnp.where(my_y == 0, n_keep, 0)

        barrier_sem = pltpu.get_barrier_semaphore()
        pl.semaphore_signal(
            barrier_sem, inc=1, device_id=peer,
            device_id_type=pl.DeviceIdType.MESH,
        )
        pl.semaphore_wait(barrier_sem, 1)

        q = n_move >> SUB_LOG2
        r = n_move & (SUB - 1)

        def compact_send(start, size):
            if size >= 4:
                def body4(i, _):
                    base = start + i * 4
                    for j in range(4):
                        send_buf[pl.ds(base + j, 1)] = x_ref[
                            pl.ds(perm_ref[n_keep + base + j], 1)
                        ]
                    return 0

                lax.fori_loop(0, size // 4, body4, 0)
            else:
                def body1(i, _):
                    send_buf[pl.ds(start + i, 1)] = x_ref[
                        pl.ds(perm_ref[n_keep + start + i], 1)
                    ]
                    return 0

                lax.fori_loop(0, size, body1, 0)

        for t in range(T // SUB):
            @pl.when(t < q)
            def _(t=t):
                off = t * SUB
                compact_send(off, SUB)
                pltpu.make_async_remote_copy(
                    src_ref=send_buf.at[pl.ds(off, SUB)],
                    dst_ref=out_ref.at[pl.ds(dst_base + off, SUB)],
                    send_sem=send_sems.at[t],
                    recv_sem=recv_sems.at[t],
                    device_id=peer,
                    device_id_type=pl.DeviceIdType.MESH,
                ).start()

        for b in range(SUB_LOG2):
            size = 1 << b

            @pl.when((r >> b) & 1 == 1)
            def _(b=b, size=size):
                off = q * SUB + (r & (size - 1))
                compact_send(off, size)
                pltpu.make_async_remote_copy(
                    src_ref=send_buf.at[pl.ds(off, size)],
                    dst_ref=out_ref.at[pl.ds(dst_base + off, size)],
                    send_sem=send_sems.at[T // SUB + b],
                    recv_sem=recv_sems.at[T // SUB + b],
                    device_id=peer,
                    device_id_type=pl.DeviceIdType.MESH,
                ).start()

        def keep4(i, _):
            base = i * 4
            for j in range(4):
                out_ref[pl.ds(keep_base + base + j, 1)] = x_ref[
                    pl.ds(perm_ref[base + j], 1)
                ]
            return 0

        lax.fori_loop(0, n_keep >> 2, keep4, 0)

        def keep1(i, _):
            out_ref[pl.ds(keep_base + i, 1)] = x_ref[pl.ds(perm_ref[i], 1)]
            return 0

        lax.fori_loop((n_keep >> 2) << 2, n_keep, keep1, 0)

        for t in range(T // SUB):
            @pl.when(t < q)
            def _(t=t):
                off = t * SUB
                desc = pltpu.make_async_remote_copy(
                    src_ref=send_buf.at[pl.ds(off, SUB)],
                    dst_ref=out_ref.at[pl.ds(recv_base + off, SUB)],
                    send_sem=send_sems.at[t],
                    recv_sem=recv_sems.at[t],
                    device_id=peer,
                    device_id_type=pl.DeviceIdType.MESH,
                )
                desc.wait_send()
                desc.wait_recv()

        for b in range(SUB_LOG2):
            size = 1 << b

            @pl.when((r >> b) & 1 == 1)
            def _(b=b, size=size):
                off = q * SUB + (r & (size - 1))
                desc = pltpu.make_async_remote_copy(
                    src_ref=send_buf.at[pl.ds(off, size)],
                    dst_ref=out_ref.at[pl.ds(recv_base + off, size)],
                    send_sem=send_sems.at[T // SUB + b],
                    recv_sem=recv_sems.at[T // SUB + b],
                    device_id=peer,
                    device_id_type=pl.DeviceIdType.MESH,
                )
                desc.wait_send()
                desc.wait_recv()

    out = pl.pallas_call(
        body,
        out_shape=jax.ShapeDtypeStruct((T, R, 128), jnp.float32),
        in_specs=[
            pl.BlockSpec(memory_space=pltpu.SMEM),
            pl.BlockSpec(memory_space=pltpu.SMEM),
            pl.BlockSpec(memory_space=pltpu.VMEM),
        ],
        out_specs=pl.BlockSpec(memory_space=pltpu.VMEM),
        scratch_shapes=[
            pltpu.VMEM((T, R, 128), jnp.float32),
            pltpu.SemaphoreType.DMA((T // SUB + SUB_LOG2,)),
            pltpu.SemaphoreType.DMA((T // SUB + SUB_LOG2,)),
        ],
        compiler_params=pltpu.CompilerParams(collective_id=0),
    )(n_keep_arr, perm, x3)
    return out.reshape(T, D)


# device time: 121486 ns/iter; 1.0816x vs baseline; 1.0816x over previous
import jax
import jax.numpy as jnp
from jax import lax
from jax.experimental import pallas as pl
from jax.experimental.pallas import tpu as pltpu

SUB = 256
SUB_LOG2 = 8
NBITS = 13


def kernel(x, dest):
    T, D = x.shape
    R = D // 128
    NCHUNK = T // SUB
    my_y_out = lax.axis_index("y")

    send = (dest != my_y_out).astype(jnp.int32)
    sc = jnp.cumsum(send)
    kc = jnp.cumsum(1 - send)
    n_keep = kc[-1]
    n_keep_arr = n_keep.astype(jnp.int32).reshape(1)
    inv = jnp.where(send == 0, kc - 1, n_keep + sc - 1).astype(jnp.int32)
    tau = jnp.searchsorted(sc, SUB * jnp.arange(1, NCHUNK + 1, dtype=jnp.int32))
    blk = jnp.concatenate(
        [jnp.zeros((1,), jnp.int32),
         jnp.minimum(tau.astype(jnp.int32) // 8 + 1, T // 8)]
    )

    def body(n_keep_ref, inv_ref, blk_ref, x_ref, out_ref, sorted_buf,
             send_sems, recv_sems, copy_sems):
        my_x = lax.axis_index("x")
        my_y = lax.axis_index("y")
        my_z = lax.axis_index("z")
        peer = (my_x, 1 - my_y, my_z)

        n_keep = n_keep_ref[0]
        n_move = T - n_keep
        q = n_move >> SUB_LOG2
        r = n_move & (SUB - 1)

        keep_base = jnp.where(my_y == 0, 0, n_move)
        dst_base = jnp.where(my_y == 0, 0, n_keep)
        recv_base = jnp.where(my_y == 0, n_keep, 0)

        barrier_sem = pltpu.get_barrier_semaphore()
        pl.semaphore_signal(
            barrier_sem, inc=1, device_id=peer,
            device_id_type=pl.DeviceIdType.MESH,
        )
        pl.semaphore_wait(barrier_sem, 1)

        def block_body(k, _):
            v = x_ref[pl.ds(8 * k, 8), :]
            for j in range(8):
                w = jnp.reshape(v[j:j + 1, :], (1, R, 128))
                pos = inv_ref[8 * k + j]
                sorted_buf[pl.ds(pos, 1)] = w
            return 0

        for t in range(NCHUNK):
            @pl.when(t < q)
            def _(t=t):
                lax.fori_loop(blk_ref[t], blk_ref[t + 1], block_body, 0)
                pltpu.make_async_remote_copy(
                    src_ref=sorted_buf.at[pl.ds(n_keep + t * SUB, SUB)],
                    dst_ref=out_ref.at[pl.ds(dst_base + t * SUB, SUB)],
                    send_sem=send_sems.at[t],
                    recv_sem=recv_sems.at[t],
                    device_id=peer,
                    device_id_type=pl.DeviceIdType.MESH,
                ).start()

        lax.fori_loop(blk_ref[q], T // 8, block_body, 0)

        for b in range(SUB_LOG2):
            size = 1 << b

            @pl.when((r >> b) & 1 == 1)
            def _(b=b, size=size):
                off = q * SUB + (r & (size - 1))
                pltpu.make_async_remote_copy(
                    src_ref=sorted_buf.at[pl.ds(n_keep + off, size)],
                    dst_ref=out_ref.at[pl.ds(dst_base + off, size)],
                    send_sem=send_sems.at[NCHUNK + b],
                    recv_sem=recv_sems.at[NCHUNK + b],
                    device_id=peer,
                    device_id_type=pl.DeviceIdType.MESH,
                ).start()

        for b in range(NBITS):
            size = 1 << b

            @pl.when((n_keep >> b) & 1 == 1)
            def _(b=b, size=size):
                off = (n_keep >> (b + 1)) << (b + 1)
                pltpu.make_async_copy(
                    sorted_buf.at[pl.ds(off, size)],
                    out_ref.at[pl.ds(keep_base + off, size)],
                    copy_sems.at[b],
                ).start()

        for t in range(NCHUNK):
            @pl.when(t < q)
            def _(t=t):
                desc = pltpu.make_async_remote_copy(
                    src_ref=sorted_buf.at[pl.ds(n_keep + t * SUB, SUB)],
                    dst_ref=out_ref.at[pl.ds(recv_base + t * SUB, SUB)],
                    send_sem=send_sems.at[t],
                    recv_sem=recv_sems.at[t],
                    device_id=peer,
                    device_id_type=pl.DeviceIdType.MESH,
                )
                desc.wait_send()
                desc.wait_recv()

        for b in range(SUB_LOG2):
            size = 1 << b

            @pl.when((r >> b) & 1 == 1)
            def _(b=b, size=size):
                off = q * SUB + (r & (size - 1))
                desc = pltpu.make_async_remote_copy(
                    src_ref=sorted_buf.at[pl.ds(n_keep + off, size)],
                    dst_ref=out_ref.at[pl.ds(recv_base + off, size)],
                    send_sem=send_sems.at[NCHUNK + b],
                    recv_sem=recv_sems.at[NCHUNK + b],
                    device_id=peer,
                    device_id_type=pl.DeviceIdType.MESH,
                )
                desc.wait_send()
                desc.wait_recv()

        for b in range(NBITS):
            size = 1 << b

            @pl.when((n_keep >> b) & 1 == 1)
            def _(b=b, size=size):
                off = (n_keep >> (b + 1)) << (b + 1)
                pltpu.make_async_copy(
                    sorted_buf.at[pl.ds(off, size)],
                    out_ref.at[pl.ds(keep_base + off, size)],
                    copy_sems.at[b],
                ).wait()

    out = pl.pallas_call(
        body,
        out_shape=jax.ShapeDtypeStruct((T, R, 128), jnp.float32),
        in_specs=[
            pl.BlockSpec(memory_space=pltpu.SMEM),
            pl.BlockSpec(memory_space=pltpu.SMEM),
            pl.BlockSpec(memory_space=pltpu.SMEM),
            pl.BlockSpec(memory_space=pltpu.VMEM),
        ],
        out_specs=pl.BlockSpec(memory_space=pltpu.VMEM),
        scratch_shapes=[
            pltpu.VMEM((T, R, 128), jnp.float32),
            pltpu.SemaphoreType.DMA((NCHUNK + SUB_LOG2,)),
            pltpu.SemaphoreType.DMA((NCHUNK + SUB_LOG2,)),
            pltpu.SemaphoreType.DMA((NBITS,)),
        ],
        compiler_params=pltpu.CompilerParams(collective_id=0),
    )(n_keep_arr, inv, blk, x)
    return out.reshape(T, D)


# device time: 119792 ns/iter; 1.0969x vs baseline; 1.0141x over previous
import jax
import jax.numpy as jnp
from jax import lax
from jax.experimental import pallas as pl
from jax.experimental.pallas import tpu as pltpu

SUB = 256
SUB_LOG2 = 8
NBITS = 13


def kernel(x, dest):
    T, D = x.shape
    R = D // 128
    NCHUNK = T // SUB
    my_y_out = lax.axis_index("y")

    send = (dest != my_y_out).astype(jnp.int32)
    s2 = send.reshape(T // 128, 128)
    c2 = jnp.cumsum(s2, axis=1)
    row_off = jnp.concatenate(
        [jnp.zeros((1,), jnp.int32), jnp.cumsum(c2[:, -1])[:-1]]
    )
    sc = (c2 + row_off[:, None]).reshape(T)
    n_keep = T - sc[-1]
    n_keep_arr = n_keep.astype(jnp.int32).reshape(1)
    inv = jnp.where(
        send == 0, jnp.arange(T, dtype=jnp.int32) - sc, n_keep + sc - 1
    ).astype(jnp.int32)
    tau = jnp.searchsorted(sc, SUB * jnp.arange(1, NCHUNK + 1, dtype=jnp.int32))
    blk = jnp.concatenate(
        [jnp.zeros((1,), jnp.int32),
         jnp.minimum(tau.astype(jnp.int32) // 8 + 1, T // 8)]
    )

    def body(n_keep_ref, inv_ref, blk_ref, x_ref, out_ref, sorted_buf,
             send_sems, recv_sems, copy_sems):
        my_x = lax.axis_index("x")
        my_y = lax.axis_index("y")
        my_z = lax.axis_index("z")
        peer = (my_x, 1 - my_y, my_z)

        n_keep = n_keep_ref[0]
        n_move = T - n_keep
        q = n_move >> SUB_LOG2
        r = n_move & (SUB - 1)

        keep_base = jnp.where(my_y == 0, 0, n_move)
        dst_base = jnp.where(my_y == 0, 0, n_keep)
        recv_base = jnp.where(my_y == 0, n_keep, 0)

        barrier_sem = pltpu.get_barrier_semaphore()
        pl.semaphore_signal(
            barrier_sem, inc=1, device_id=peer,
            device_id_type=pl.DeviceIdType.MESH,
        )
        pl.semaphore_wait(barrier_sem, 1)

        def block_body(k, _):
            v = x_ref[pl.ds(8 * k, 8), :]
            for j in range(8):
                w = jnp.reshape(v[j:j + 1, :], (1, R, 128))
                pos = inv_ref[8 * k + j]
                sorted_buf[pl.ds(pos, 1)] = w
            return 0

        for t in range(NCHUNK):
            @pl.when(t < q)
            def _(t=t):
                lax.fori_loop(blk_ref[t], blk_ref[t + 1], block_body, 0)
                pltpu.make_async_remote_copy(
                    src_ref=sorted_buf.at[pl.ds(n_keep + t * SUB, SUB)],
                    dst_ref=out_ref.at[pl.ds(dst_base + t * SUB, SUB)],
                    send_sem=send_sems.at[t],
                    recv_sem=recv_sems.at[t],
                    device_id=peer,
                    device_id_type=pl.DeviceIdType.MESH,
                ).start()

        lax.fori_loop(blk_ref[q], T // 8, block_body, 0)

        for b in range(SUB_LOG2):
            size = 1 << b

            @pl.when((r >> b) & 1 == 1)
            def _(b=b, size=size):
                off = q * SUB + (r & (size - 1))
                pltpu.make_async_remote_copy(
                    src_ref=sorted_buf.at[pl.ds(n_keep + off, size)],
                    dst_ref=out_ref.at[pl.ds(dst_base + off, size)],
                    send_sem=send_sems.at[NCHUNK + b],
                    recv_sem=recv_sems.at[NCHUNK + b],
                    device_id=peer,
                    device_id_type=pl.DeviceIdType.MESH,
                ).start()

        for b in range(NBITS):
            size = 1 << b

            @pl.when((n_keep >> b) & 1 == 1)
            def _(b=b, size=size):
                off = (n_keep >> (b + 1)) << (b + 1)
                pltpu.make_async_copy(
                    sorted_buf.at[pl.ds(off, size)],
                    out_ref.at[pl.ds(keep_base + off, size)],
                    copy_sems.at[b],
                ).start()

        for t in range(NCHUNK):
            @pl.when(t < q)
            def _(t=t):
                desc = pltpu.make_async_remote_copy(
                    src_ref=sorted_buf.at[pl.ds(n_keep + t * SUB, SUB)],
                    dst_ref=out_ref.at[pl.ds(recv_base + t * SUB, SUB)],
                    send_sem=send_sems.at[t],
                    recv_sem=recv_sems.at[t],
                    device_id=peer,
                    device_id_type=pl.DeviceIdType.MESH,
                )
                desc.wait_send()
                desc.wait_recv()

        for b in range(SUB_LOG2):
            size = 1 << b

            @pl.when((r >> b) & 1 == 1)
            def _(b=b, size=size):
                off = q * SUB + (r & (size - 1))
                desc = pltpu.make_async_remote_copy(
                    src_ref=sorted_buf.at[pl.ds(n_keep + off, size)],
                    dst_ref=out_ref.at[pl.ds(recv_base + off, size)],
                    send_sem=send_sems.at[NCHUNK + b],
                    recv_sem=recv_sems.at[NCHUNK + b],
                    device_id=peer,
                    device_id_type=pl.DeviceIdType.MESH,
                )
                desc.wait_send()
                desc.wait_recv()

        for b in range(NBITS):
            size = 1 << b

            @pl.when((n_keep >> b) & 1 == 1)
            def _(b=b, size=size):
                off = (n_keep >> (b + 1)) << (b + 1)
                pltpu.make_async_copy(
                    sorted_buf.at[pl.ds(off, size)],
                    out_ref.at[pl.ds(keep_base + off, size)],
                    copy_sems.at[b],
                ).wait()

    out = pl.pallas_call(
        body,
        out_shape=jax.ShapeDtypeStruct((T, R, 128), jnp.float32),
        in_specs=[
            pl.BlockSpec(memory_space=pltpu.SMEM),
            pl.BlockSpec(memory_space=pltpu.SMEM),
            pl.BlockSpec(memory_space=pltpu.SMEM),
            pl.BlockSpec(memory_space=pltpu.VMEM),
        ],
        out_specs=pl.BlockSpec(memory_space=pltpu.VMEM),
        scratch_shapes=[
            pltpu.VMEM((T, R, 128), jnp.float32),
            pltpu.SemaphoreType.DMA((NCHUNK + SUB_LOG2,)),
            pltpu.SemaphoreType.DMA((NCHUNK + SUB_LOG2,)),
            pltpu.SemaphoreType.DMA((NBITS,)),
        ],
        compiler_params=pltpu.CompilerParams(collective_id=0),
    )(n_keep_arr, inv, blk, x)
    return out.reshape(T, D)
